# baseline (speedup 1.0000x reference)
import functools

import jax
import jax.numpy as jnp
from jax import lax
from jax.experimental import pallas as pl
from jax.experimental.pallas import tpu as pltpu
from jax.experimental.pallas import tpu_sc as plsc

N = 10000
DH = 128
DOUT = 64
E = 320000

NC = 2
NS = 16
NW = NC * NS
BLK = 128
G = 2
H = 2
NBLK = 80
HBLK = NBLK // H
E_PAD = NW * BLK * NBLK
N_PAD = N + 112
ROWS_PER_TILE = N_PAD // NS

_mesh = plsc.VectorSubcoreMesh(core_axis_name="c", subcore_axis_name="s")


@functools.partial(
    pl.kernel,
    out_type=jax.ShapeDtypeStruct((NC, N_PAD, DH), jnp.float32),
    mesh=_mesh,
    scratch_types=[
        pltpu.VMEM((NBLK, 1, BLK), jnp.int32),
        pltpu.VMEM((BLK, DH), jnp.float32),
        pltpu.VMEM_SHARED((N_PAD, DH), jnp.float32),
    ],
)
def _sc_degree(dst3_hbm, zeros_hbm, ones_hbm, out_hbm,
               didx_all, ones_v, acc_sh):
    c = lax.axis_index("c")
    s = lax.axis_index("s")
    wid = c * NS + s
    r0 = s * ROWS_PER_TILE
    pltpu.sync_copy(ones_hbm, ones_v)
    pltpu.sync_copy(dst3_hbm.at[pl.ds(wid * NBLK, NBLK)], didx_all)
    pltpu.sync_copy(zeros_hbm.at[pl.ds(r0, ROWS_PER_TILE)],
                    acc_sh.at[pl.ds(r0, ROWS_PER_TILE)])
    plsc.subcore_barrier()

    def group(i, carry):
        pltpu.sync_copy(ones_v, acc_sh.at[didx_all.at[i, 0]], add=True)
        return carry

    lax.fori_loop(0, NBLK, group, 0)
    plsc.subcore_barrier()
    pltpu.sync_copy(acc_sh.at[pl.ds(r0, ROWS_PER_TILE)],
                    out_hbm.at[c, pl.ds(r0, ROWS_PER_TILE)])


@functools.partial(
    pl.kernel,
    out_type=jax.ShapeDtypeStruct((NC, N_PAD, DH), jnp.float32),
    mesh=_mesh,
    scratch_types=[
        pltpu.VMEM((HBLK * BLK,), jnp.int32),
        pltpu.VMEM((HBLK, 1, BLK), jnp.int32),
    ] + [pltpu.VMEM((BLK, DH), jnp.float32)] * G
      + [pltpu.VMEM_SHARED((N_PAD, DH), jnp.float32)]
      + [pltpu.SemaphoreType.DMA] * G,
)
def _sc_scatter(hp_hbm, src_hbm, dst3_hbm, zeros_hbm, out_hbm,
                sidx_all, didx_all, *bufs):
    rows = bufs[:G]
    acc_sh = bufs[G]
    gsem = bufs[G + 1:G + 1 + G]
    c = lax.axis_index("c")
    s = lax.axis_index("s")
    wid = c * NS + s
    r0 = s * ROWS_PER_TILE
    pltpu.sync_copy(zeros_hbm.at[pl.ds(r0, ROWS_PER_TILE)],
                    acc_sh.at[pl.ds(r0, ROWS_PER_TILE)])
    plsc.subcore_barrier()

    def gather_start(j, b):
        pltpu.async_copy(hp_hbm.at[sidx_all.at[pl.ds(j * BLK, BLK)]],
                         rows[b], gsem[b])

    for h in range(H):
        hb0 = wid * NBLK + h * HBLK
        pltpu.sync_copy(src_hbm.at[pl.ds(hb0 * BLK, HBLK * BLK)], sidx_all)
        pltpu.sync_copy(dst3_hbm.at[pl.ds(hb0, HBLK)], didx_all)
        for b in range(G):
            gather_start(b, b)

        def group(i, carry):
            j0 = i * G
            for b in range(G):
                pltpu.make_async_copy(zeros_hbm.at[pl.ds(0, BLK)],
                                      rows[b], gsem[b]).wait()
                pltpu.sync_copy(rows[b], acc_sh.at[didx_all.at[j0 + b, 0]],
                                add=True)
                nj = j0 + b + G

                @pl.when(nj < HBLK)
                def _():
                    gather_start(nj, b)
            return carry

        lax.fori_loop(0, HBLK // G, group, 0)
    plsc.subcore_barrier()
    pltpu.sync_copy(acc_sh.at[pl.ds(r0, ROWS_PER_TILE)],
                    out_hbm.at[c, pl.ds(r0, ROWS_PER_TILE)])


def _dis_from_degp(degp_ref):
    deg = degp_ref[0][:N, 0:1] + degp_ref[1][:N, 0:1] + 1.0
    return lax.rsqrt(deg)


def _pool(x, gwt, gb, lp, lpb):
    logit = jnp.sum(x * gwt, axis=1, keepdims=True) + gb
    e = jnp.exp(logit - jnp.max(logit))
    pool = jnp.sum(x * e, axis=0, keepdims=True) / jnp.sum(e)
    return jnp.dot(pool, lp, preferred_element_type=jnp.float32) + lpb


def _tc0_body(x_ref, degp_ref, W_ref, gwt_ref, gb_ref, lp_ref, lpb_ref,
              hp_ref, pool_ref):
    x = x_ref[...]
    dis = _dis_from_degp(degp_ref)
    pool_ref[...] = _pool(x, gwt_ref[...], gb_ref[...], lp_ref[...], lpb_ref[...])
    hp_ref[...] = jnp.dot(x, W_ref[...], preferred_element_type=jnp.float32) * dis


def _tcmid_body(aggp_ref, hp_prev_ref, degp_ref, b_ref, W_ref,
                gwt_ref, gb_ref, lp_ref, lpb_ref, hp_ref, pool_ref):
    dis = _dis_from_degp(degp_ref)
    agg = aggp_ref[0][:N, :] + aggp_ref[1][:N, :] + hp_prev_ref[...]
    x = agg * dis + b_ref[...]
    pool_ref[...] = _pool(x, gwt_ref[...], gb_ref[...], lp_ref[...], lpb_ref[...])
    hp_ref[...] = jnp.dot(x, W_ref[...], preferred_element_type=jnp.float32) * dis


def _tcfin_body(aggp_ref, hp_prev_ref, degp_ref, b_ref,
                gwt_ref, gb_ref, cw_ref, cb_ref,
                p0_ref, p1_ref, p2_ref, h0_ref, beta_ref, risk_ref):
    dis = _dis_from_degp(degp_ref)
    agg = aggp_ref[0][:N, :] + aggp_ref[1][:N, :] + hp_prev_ref[...]
    x = agg * dis + b_ref[...]
    p3 = _pool(x, gwt_ref[...], gb_ref[...], cw_ref[...], cb_ref[...])
    out = (p0_ref[...] + p1_ref[...] + p2_ref[...] + p3) * 0.25
    val = jnp.sum(out * beta_ref[...])
    risk_ref[...] = jnp.exp(h0_ref[...] + val)


def _tc_stage0(x, degp, W, gwt, gb, lp, lpb):
    return pl.pallas_call(
        _tc0_body,
        out_shape=[jax.ShapeDtypeStruct((N, DH), jnp.float32),
                   jax.ShapeDtypeStruct((1, DOUT), jnp.float32)],
    )(x, degp, W, gwt, gb, lp, lpb)


def _tc_stage(aggp, hp_prev, degp, b, W, gwt, gb, lp, lpb):
    return pl.pallas_call(
        _tcmid_body,
        out_shape=[jax.ShapeDtypeStruct((N, DH), jnp.float32),
                   jax.ShapeDtypeStruct((1, DOUT), jnp.float32)],
    )(aggp, hp_prev, degp, b, W, gwt, gb, lp, lpb)


def _tc_final(aggp, hp_prev, degp, b, gwt, gb, cw, cb, p0, p1, p2, h0, beta):
    return pl.pallas_call(
        _tcfin_body,
        out_shape=jax.ShapeDtypeStruct((1, 1), jnp.float32),
    )(aggp, hp_prev, degp, b, gwt, gb, cw, cb, p0, p1, p2, h0, beta)


def kernel(x, edge_index, W0, b0, W1, b1, W2, b2, gw0, gb0, gw1, gb1,
           gw2, gb2, gw3, gb3, lpw0, lpb0, lpw1, lpb1, lpw2, lpb2,
           cw, cb, h0, beta):
    src = edge_index[0]
    dst = edge_index[1]
    pad = E_PAD - E
    srcp = jnp.concatenate([src, jnp.zeros((pad,), jnp.int32)])
    trash = N + (jnp.arange(pad, dtype=jnp.int32) % (N_PAD - N))
    dst3 = jnp.concatenate([dst, trash]).reshape(NW * NBLK, 1, BLK)
    zrow = jnp.zeros((N_PAD, DH), jnp.float32)
    ones_blk = jnp.ones((BLK, DH), jnp.float32)

    degp = _sc_degree(dst3, zrow, ones_blk)

    hp0, p0 = _tc_stage0(x, degp, W0, gw0.reshape(1, DH), gb0.reshape(1, 1),
                         lpw0, lpb0.reshape(1, DOUT))
    agg0 = _sc_scatter(hp0, srcp, dst3, zrow)
    hp1, p1 = _tc_stage(agg0, hp0, degp, b0.reshape(1, DH), W1,
                        gw1.reshape(1, DH), gb1.reshape(1, 1),
                        lpw1, lpb1.reshape(1, DOUT))
    agg1 = _sc_scatter(hp1, srcp, dst3, zrow)
    hp2, p2 = _tc_stage(agg1, hp1, degp, b1.reshape(1, DH), W2,
                        gw2.reshape(1, DH), gb2.reshape(1, 1),
                        lpw2, lpb2.reshape(1, DOUT))
    agg2 = _sc_scatter(hp2, srcp, dst3, zrow)
    risk = _tc_final(agg2, hp2, degp, b2.reshape(1, DH),
                     gw3.reshape(1, DH), gb3.reshape(1, 1), cw,
                     cb.reshape(1, DOUT), p0, p1, p2,
                     h0.reshape(1, 1), beta.reshape(1, DOUT))
    return risk.reshape(1)


# speedup vs baseline: 1.0695x; 1.0695x over previous
import functools

import jax
import jax.numpy as jnp
from jax import lax
from jax.experimental import pallas as pl
from jax.experimental.pallas import tpu as pltpu
from jax.experimental.pallas import tpu_sc as plsc

N = 10000
DH = 128
DOUT = 64
E = 320000

NC = 2
NS = 16
NW = NC * NS
BLK = 128
G = 2
NBLK = 80
HBLK = 32
NB0 = 128
NB1 = 32
E_PAD = NW * BLK * NBLK
N_PAD = N + 112
ROWS_PER_TILE = N_PAD // NS

_mesh = plsc.VectorSubcoreMesh(core_axis_name="c", subcore_axis_name="s")


@functools.partial(
    pl.kernel,
    out_type=jax.ShapeDtypeStruct((NC, N_PAD, DH), jnp.float32),
    mesh=_mesh,
    scratch_types=[
        pltpu.VMEM((NBLK, 1, BLK), jnp.int32),
        pltpu.VMEM((BLK, DH), jnp.float32),
        pltpu.VMEM_SHARED((N_PAD, DH), jnp.float32),
    ],
)
def _sc_degree(dst3_hbm, zeros_hbm, ones_hbm, out_hbm,
               didx_all, ones_v, acc_sh):
    c = lax.axis_index("c")
    s = lax.axis_index("s")
    wid = c * NS + s
    r0 = s * ROWS_PER_TILE
    pltpu.sync_copy(ones_hbm, ones_v)
    pltpu.sync_copy(dst3_hbm.at[pl.ds(wid * NBLK, NBLK)], didx_all)
    pltpu.sync_copy(zeros_hbm.at[pl.ds(r0, ROWS_PER_TILE)],
                    acc_sh.at[pl.ds(r0, ROWS_PER_TILE)])
    plsc.subcore_barrier()

    def group(i, carry):
        pltpu.sync_copy(ones_v, acc_sh.at[didx_all.at[i, 0]], add=True)
        return carry

    lax.fori_loop(0, NBLK, group, 0)
    plsc.subcore_barrier()
    pltpu.sync_copy(acc_sh.at[pl.ds(r0, ROWS_PER_TILE)],
                    out_hbm.at[c, pl.ds(r0, ROWS_PER_TILE)])


@functools.partial(
    pl.kernel,
    out_type=jax.ShapeDtypeStruct((NC, N_PAD, DH), jnp.float32),
    mesh=_mesh,
    scratch_types=[
        pltpu.VMEM((HBLK * BLK,), jnp.int32),
        pltpu.VMEM((HBLK, 1, BLK), jnp.int32),
    ] + [pltpu.VMEM((BLK, DH), jnp.float32)] * G
      + [pltpu.VMEM_SHARED((N_PAD, DH), jnp.float32)]
      + [pltpu.SemaphoreType.DMA] * G,
)
def _sc_scatter(hp_hbm, src_hbm, dst3_hbm, zeros_hbm, out_hbm,
                sidx_all, didx_all, *bufs):
    rows = bufs[:G]
    acc_sh = bufs[G]
    gsem = bufs[G + 1:G + 1 + G]
    c = lax.axis_index("c")
    s = lax.axis_index("s")
    wid = c * NS + s
    r0 = s * ROWS_PER_TILE
    pltpu.sync_copy(zeros_hbm.at[pl.ds(r0, ROWS_PER_TILE)],
                    acc_sh.at[pl.ds(r0, ROWS_PER_TILE)])
    plsc.subcore_barrier()

    def gather_start(j, b):
        pltpu.async_copy(hp_hbm.at[sidx_all.at[pl.ds(j * BLK, BLK)]],
                         rows[b], gsem[b])

    def pipeline(hb0):
        pltpu.sync_copy(src_hbm.at[pl.ds(hb0 * BLK, HBLK * BLK)], sidx_all)
        pltpu.sync_copy(dst3_hbm.at[pl.ds(hb0, HBLK)], didx_all)
        for b in range(G):
            gather_start(b, b)

        def group(i, carry):
            j0 = i * G
            for b in range(G):
                pltpu.make_async_copy(zeros_hbm.at[pl.ds(0, BLK)],
                                      rows[b], gsem[b]).wait()
                pltpu.sync_copy(rows[b], acc_sh.at[didx_all.at[j0 + b, 0]],
                                add=True)
                nj = j0 + b + G

                @pl.when(nj < HBLK)
                def _():
                    gather_start(nj, b)
            return carry

        lax.fori_loop(0, HBLK // G, group, 0)

    @pl.when(c == 0)
    def _():
        for h in range(NB0 // HBLK):
            pipeline(s * NB0 + h * HBLK)

    @pl.when(c == 1)
    def _():
        for h in range(NB1 // HBLK):
            pipeline(NS * NB0 + s * NB1 + h * HBLK)

    plsc.subcore_barrier()
    pltpu.sync_copy(acc_sh.at[pl.ds(r0, ROWS_PER_TILE)],
                    out_hbm.at[c, pl.ds(r0, ROWS_PER_TILE)])


def _dis_from_degp(degp_ref):
    deg = degp_ref[0][:N, 0:1] + degp_ref[1][:N, 0:1] + 1.0
    return lax.rsqrt(deg)


def _pool(x, gwt, gb, lp, lpb):
    logit = jnp.sum(x * gwt, axis=1, keepdims=True) + gb
    e = jnp.exp(logit - jnp.max(logit))
    pool = jnp.sum(x * e, axis=0, keepdims=True) / jnp.sum(e)
    return jnp.dot(pool, lp, preferred_element_type=jnp.float32) + lpb


def _tc0_body(x_ref, degp_ref, W_ref, gwt_ref, gb_ref, lp_ref, lpb_ref,
              hp_ref, pool_ref):
    x = x_ref[...]
    dis = _dis_from_degp(degp_ref)
    pool_ref[...] = _pool(x, gwt_ref[...], gb_ref[...], lp_ref[...], lpb_ref[...])
    hp_ref[...] = jnp.dot(x, W_ref[...], preferred_element_type=jnp.float32) * dis


def _tcmid_body(aggp_ref, hp_prev_ref, degp_ref, b_ref, W_ref,
                gwt_ref, gb_ref, lp_ref, lpb_ref, hp_ref, pool_ref):
    dis = _dis_from_degp(degp_ref)
    agg = aggp_ref[0][:N, :] + aggp_ref[1][:N, :] + hp_prev_ref[...]
    x = agg * dis + b_ref[...]
    pool_ref[...] = _pool(x, gwt_ref[...], gb_ref[...], lp_ref[...], lpb_ref[...])
    hp_ref[...] = jnp.dot(x, W_ref[...], preferred_element_type=jnp.float32) * dis


def _tcfin_body(aggp_ref, hp_prev_ref, degp_ref, b_ref,
                gwt_ref, gb_ref, cw_ref, cb_ref,
                p0_ref, p1_ref, p2_ref, h0_ref, beta_ref, risk_ref):
    dis = _dis_from_degp(degp_ref)
    agg = aggp_ref[0][:N, :] + aggp_ref[1][:N, :] + hp_prev_ref[...]
    x = agg * dis + b_ref[...]
    p3 = _pool(x, gwt_ref[...], gb_ref[...], cw_ref[...], cb_ref[...])
    out = (p0_ref[...] + p1_ref[...] + p2_ref[...] + p3) * 0.25
    val = jnp.sum(out * beta_ref[...])
    risk_ref[...] = jnp.exp(h0_ref[...] + val)


def _tc_stage0(x, degp, W, gwt, gb, lp, lpb):
    return pl.pallas_call(
        _tc0_body,
        out_shape=[jax.ShapeDtypeStruct((N, DH), jnp.float32),
                   jax.ShapeDtypeStruct((1, DOUT), jnp.float32)],
    )(x, degp, W, gwt, gb, lp, lpb)


def _tc_stage(aggp, hp_prev, degp, b, W, gwt, gb, lp, lpb):
    return pl.pallas_call(
        _tcmid_body,
        out_shape=[jax.ShapeDtypeStruct((N, DH), jnp.float32),
                   jax.ShapeDtypeStruct((1, DOUT), jnp.float32)],
    )(aggp, hp_prev, degp, b, W, gwt, gb, lp, lpb)


def _tc_final(aggp, hp_prev, degp, b, gwt, gb, cw, cb, p0, p1, p2, h0, beta):
    return pl.pallas_call(
        _tcfin_body,
        out_shape=jax.ShapeDtypeStruct((1, 1), jnp.float32),
    )(aggp, hp_prev, degp, b, gwt, gb, cw, cb, p0, p1, p2, h0, beta)


def kernel(x, edge_index, W0, b0, W1, b1, W2, b2, gw0, gb0, gw1, gb1,
           gw2, gb2, gw3, gb3, lpw0, lpb0, lpw1, lpb1, lpw2, lpb2,
           cw, cb, h0, beta):
    src = edge_index[0]
    dst = edge_index[1]
    pad = E_PAD - E
    srcp = jnp.concatenate([src, jnp.zeros((pad,), jnp.int32)])
    trash = N + (jnp.arange(pad, dtype=jnp.int32) % (N_PAD - N))
    dst3 = jnp.concatenate([dst, trash]).reshape(NW * NBLK, 1, BLK)
    zrow = jnp.zeros((N_PAD, DH), jnp.float32)
    ones_blk = jnp.ones((BLK, DH), jnp.float32)

    degp = _sc_degree(dst3, zrow, ones_blk)

    hp0, p0 = _tc_stage0(x, degp, W0, gw0.reshape(1, DH), gb0.reshape(1, 1),
                         lpw0, lpb0.reshape(1, DOUT))
    agg0 = _sc_scatter(hp0, srcp, dst3, zrow)
    hp1, p1 = _tc_stage(agg0, hp0, degp, b0.reshape(1, DH), W1,
                        gw1.reshape(1, DH), gb1.reshape(1, 1),
                        lpw1, lpb1.reshape(1, DOUT))
    agg1 = _sc_scatter(hp1, srcp, dst3, zrow)
    hp2, p2 = _tc_stage(agg1, hp1, degp, b1.reshape(1, DH), W2,
                        gw2.reshape(1, DH), gb2.reshape(1, 1),
                        lpw2, lpb2.reshape(1, DOUT))
    agg2 = _sc_scatter(hp2, srcp, dst3, zrow)
    risk = _tc_final(agg2, hp2, degp, b2.reshape(1, DH),
                     gw3.reshape(1, DH), gb3.reshape(1, 1), cw,
                     cb.reshape(1, DOUT), p0, p1, p2,
                     h0.reshape(1, 1), beta.reshape(1, DOUT))
    return risk.reshape(1)
